# baseline (speedup 1.0000x reference)
import functools

import jax
import jax.numpy as jnp
import numpy as np
from jax import lax
from jax.experimental import pallas as pl
from jax.experimental.pallas import tpu as pltpu
from jax.experimental.pallas import tpu_sc as plsc

_F = 50
_P = _F * _F
_PN = 2560
_INC = 512
_K9 = _INC * 9
_NANC = 9
_IMG = 800.0
_MIN_SIZE = 16.0
_NMS_THRESH = 0.7
_PRE_NMS = 6000
_POST_NMS = 300
_NBLK = 512

_NCAND = _NANC * _PN
_NTILES = 16
_PER_TILE = _NCAND // _NTILES
_NVREG = _PER_TILE // 16
_OUTW = 304
_NEG_INF = float("-inf")


def _conv_block(w1_ref, patches_ref, wrc_ref, brc_ref, anc_ref, dec_ref,
                key_ref):
    x = lax.dot_general(
        w1_ref[...], patches_ref[...],
        (((1,), (0,)), ((), ())),
        precision=lax.Precision.DEFAULT,
        preferred_element_type=jnp.float32)
    out = lax.dot_general(
        wrc_ref[...], x,
        (((1,), (0,)), ((), ())),
        precision=lax.Precision.DEFAULT,
        preferred_element_type=jnp.float32)
    out = out + brc_ref[...]

    dy = out[0:9]
    dx = out[9:18]
    dh = out[18:27]
    dw = out[27:36]
    obj = out[36:45]

    y1a = anc_ref[0:9]
    x1a = anc_ref[9:18]
    y2a = anc_ref[18:27]
    x2a = anc_ref[27:36]
    anc_h = y2a - y1a
    anc_w = x2a - x1a
    anc_cy = y1a + 0.5 * anc_h
    anc_cx = x1a + 0.5 * anc_w

    cy = dy * anc_h + anc_cy
    cx = dx * anc_w + anc_cx
    h = jnp.exp(dh) * anc_h
    w = jnp.exp(dw) * anc_w
    y1 = jnp.clip(cy - 0.5 * h, 0.0, _IMG)
    x1 = jnp.clip(cx - 0.5 * w, 0.0, _IMG)
    y2 = jnp.clip(cy + 0.5 * h, 0.0, _IMG)
    x2 = jnp.clip(cx + 0.5 * w, 0.0, _IMG)

    hs = y2 - y1
    ws = x2 - x1
    scr = jnp.where(jnp.logical_and(hs >= _MIN_SIZE, ws >= _MIN_SIZE),
                    obj, _NEG_INF)
    col = pl.program_id(0) * _NBLK + lax.broadcasted_iota(
        jnp.int32, (_NANC, _NBLK), 1)
    scr = jnp.where(col < _P, scr, _NEG_INF)

    dec_ref[0:9] = y1
    dec_ref[9:18] = x1
    dec_ref[18:27] = y2
    dec_ref[27:36] = x2
    dec_ref[36:45] = scr
    dec_ref[45:48] = jnp.zeros((3, _NBLK), jnp.float32)

    bits = lax.bitcast_convert_type(scr, jnp.int32)
    key = jnp.where(bits >= 0, bits,
                    jnp.bitwise_xor(bits, jnp.int32(0x7FFFFFFF)))
    key_ref[0:9] = key
    key_ref[9:16] = jnp.zeros((7, _NBLK), jnp.int32)


def _conv_heads(features, W1, b1, Wr, br, Wc, bc, anchors):
    f = features[0]
    fpad = jnp.pad(f, ((0, 0), (1, 1), (1, 1)))
    taps = [fpad[:, ky:ky + _F, kx:kx + _F].reshape(_INC, _P)
            for ky in range(3) for kx in range(3)]
    patches = jnp.concatenate(taps, axis=0)
    patches = jnp.pad(patches, ((0, 0), (0, _PN - _P)))

    w1mat = jnp.transpose(W1, (0, 2, 3, 1)).reshape(_INC, _K9)

    a = np.arange(_NANC)
    row_sel_r = np.concatenate([4 * a + c for c in range(4)])
    row_sel_c = 2 * a + 1
    wr2 = Wr[:, :, 0, 0]
    wc2 = Wc[:, :, 0, 0]
    wrc = jnp.concatenate([wr2[row_sel_r], wc2[row_sel_c]], axis=0)
    wrc = jnp.pad(wrc, ((0, 64 - 45), (0, 0)))
    bsel = jnp.concatenate([br[row_sel_r], bc[row_sel_c]], axis=0)
    brc = wrc @ b1 + jnp.pad(bsel, (0, 64 - 45))
    brc_b = jnp.broadcast_to(brc[:, None], (64, _PN))

    anc = anchors.reshape(_P, _NANC, 4).transpose(2, 1, 0).reshape(36, _P)
    anc = jnp.pad(anc, ((0, 4), (0, _PN - _P)))

    dec = pl.pallas_call(
        _conv_block,
        grid=(_PN // _NBLK,),
        in_specs=[
            pl.BlockSpec((_INC, _K9), lambda i: (0, 0)),
            pl.BlockSpec((_K9, _NBLK), lambda i: (0, i)),
            pl.BlockSpec((64, _INC), lambda i: (0, 0)),
            pl.BlockSpec((64, _NBLK), lambda i: (0, i)),
            pl.BlockSpec((40, _NBLK), lambda i: (0, i)),
        ],
        out_specs=[pl.BlockSpec((48, _NBLK), lambda i: (0, i)),
                   pl.BlockSpec((16, _NBLK), lambda i: (0, i))],
        out_shape=[jax.ShapeDtypeStruct((48, _PN), jnp.float32),
                   jax.ShapeDtypeStruct((16, _PN), jnp.int32)],
    )(w1mat, patches, wrc, brc_b, anc)
    return dec


def _bc_f(x):
    return lax.broadcast_in_dim(jnp.float32(x) if isinstance(x, float) else x,
                                (16,), ())


def _bc_i(x):
    return lax.broadcast_in_dim(x, (16,), ())


def _nms_sc_body(scr_hbm, y1_hbm, x1_hbm, y2_hbm, x2_hbm, key_hbm, out_hbm,
                 sc_v, y1_v, x1_v, y2_v, x2_v, key_v, out_v, pub_v, tmp_v,
                 stage_v, stage):
    wid = lax.axis_index("s")
    base = wid * _PER_TILE
    iota = lax.iota(jnp.int32, 16)

    pltpu.sync_copy(scr_hbm.at[pl.ds(base, _PER_TILE)], sc_v)
    pltpu.sync_copy(y1_hbm.at[pl.ds(base, _PER_TILE)], y1_v)
    pltpu.sync_copy(x1_hbm.at[pl.ds(base, _PER_TILE)], x1_v)
    pltpu.sync_copy(y2_hbm.at[pl.ds(base, _PER_TILE)], y2_v)
    pltpu.sync_copy(x2_hbm.at[pl.ds(base, _PER_TILE)], x2_v)
    pltpu.sync_copy(key_hbm.at[pl.ds(base, _PER_TILE)], key_v)

    def zero_out(j, carry):
        off = pl.multiple_of(j * 16, 16)
        out_v[pl.ds(off, 16)] = jnp.zeros((16,), jnp.float32)
        return carry

    lax.fori_loop(0, 4 * _OUTW // 16, zero_out, 0)

    iota_f = iota.astype(jnp.float32)

    def allred(vec, op):
        for s in (8, 4, 2, 1):
            tmp_v[pl.ds(0, 16)] = vec
            g = plsc.load_gather(
                tmp_v, [jnp.bitwise_xor(iota, _bc_i(jnp.int32(s)))])
            vec = op(vec, g)
        return vec

    def global_count_ge(t2v, boff):
        def count_vreg(j, acc):
            off = pl.multiple_of(j * 16, 16)
            k = key_v[pl.ds(off, 16)]
            return acc + jnp.where(k >= t2v, _bc_f(jnp.float32(1.0)),
                                   _bc_f(jnp.float32(0.0)))

        acc = lax.fori_loop(0, _NVREG, count_vreg,
                            jnp.zeros((16,), jnp.float32))
        pub_v[pl.ds(0, 16)] = allred(acc, jnp.add)
        pltpu.sync_copy(pub_v.at[pl.ds(0, 8)],
                        stage.at[pl.ds(wid * 8 + boff, 8)])
        plsc.subcore_barrier()
        pltpu.sync_copy(stage.at[pl.ds(boff, 128)], stage_v)
        counts = plsc.load_gather(stage_v, [iota * 8])
        return allred(counts, jnp.add)

    kf = _bc_f(jnp.float32(float(_PRE_NMS)))
    t = jnp.where(global_count_ge(_bc_i(jnp.int32(0)), 0) >= kf,
                  _bc_i(jnp.int32(0)), _bc_i(jnp.int32(-2147483648)))
    for b in range(30, -1, -1):
        t2 = t + _bc_i(jnp.int32(1 << b))
        t = jnp.where(global_count_ge(t2, 128 * ((31 - b) % 2)) >= kf, t2, t)

    def init_live(j, carry):
        bv, bi = carry
        off = pl.multiple_of(j * 16, 16)
        k = key_v[pl.ds(off, 16)]
        s = sc_v[pl.ds(off, 16)]
        s2 = jnp.where(k >= t, s, _bc_f(_NEG_INF))
        sc_v[pl.ds(off, 16)] = s2
        pred = s2 > bv
        bv = jnp.where(pred, s2, bv)
        bi = jnp.where(pred, _bc_f(off.astype(jnp.float32)) + iota_f, bi)
        return (bv, bi)

    bv0, bi0 = lax.fori_loop(
        0, _NVREG, init_live,
        (jnp.full((16,), _NEG_INF, jnp.float32),
         jnp.zeros((16,), jnp.float32)))
    plsc.subcore_barrier()

    def round_body(r, st):
        cnt_v, bv, bi = st
        mv = allred(bv, jnp.maximum)
        sel = jnp.where(bv >= mv, bi, _bc_f(jnp.float32(float(1 << 30))))
        liv = allred(sel, jnp.minimum).astype(jnp.int32)
        ly1 = plsc.load_gather(y1_v, [liv])
        lx1 = plsc.load_gather(x1_v, [liv])
        ly2 = plsc.load_gather(y2_v, [liv])
        lx2 = plsc.load_gather(x2_v, [liv])
        pub = jnp.where(iota == _bc_i(jnp.int32(0)), mv,
              jnp.where(iota == _bc_i(jnp.int32(1)), ly1,
              jnp.where(iota == _bc_i(jnp.int32(2)), lx1,
              jnp.where(iota == _bc_i(jnp.int32(3)), ly2, lx2))))
        pub_v[pl.ds(0, 16)] = pub
        boff = jnp.bitwise_and(r, jnp.int32(1)) * 128
        pltpu.sync_copy(pub_v.at[pl.ds(0, 8)],
                        stage.at[pl.ds(wid * 8 + boff, 8)])
        plsc.subcore_barrier()
        pltpu.sync_copy(stage.at[pl.ds(boff, 128)], stage_v)

        keys = plsc.load_gather(stage_v, [iota * 8])
        gmaxv = allred(keys, jnp.maximum)
        alive = gmaxv > _bc_f(_NEG_INF)
        wsel = jnp.where(keys >= gmaxv, iota_f, _bc_f(jnp.float32(99.0)))
        woff = allred(wsel, jnp.minimum).astype(jnp.int32) * _bc_i(jnp.int32(8))
        wy1 = plsc.load_gather(stage_v, [woff + _bc_i(jnp.int32(1))])
        wx1 = plsc.load_gather(stage_v, [woff + _bc_i(jnp.int32(2))])
        wy2 = plsc.load_gather(stage_v, [woff + _bc_i(jnp.int32(3))])
        wx2 = plsc.load_gather(stage_v, [woff + _bc_i(jnp.int32(4))])

        outv = jnp.where(iota == _bc_i(jnp.int32(0)), wy1,
               jnp.where(iota == _bc_i(jnp.int32(1)), wx1,
               jnp.where(iota == _bc_i(jnp.int32(2)), wy2, wx2)))
        omask = jnp.logical_and(iota < _bc_i(jnp.int32(4)), alive)
        plsc.store_scatter(out_v, [cnt_v + iota * _OUTW], outv, mask=omask)

        warea = (wx2 - wx1 + 1.0) * (wy2 - wy1 + 1.0)

        def sup_amax(j, carry):
            nbv, nbi = carry
            off = pl.multiple_of(j * 16, 16)
            cy1 = y1_v[pl.ds(off, 16)]
            cx1 = x1_v[pl.ds(off, 16)]
            cy2 = y2_v[pl.ds(off, 16)]
            cx2 = x2_v[pl.ds(off, 16)]
            s = sc_v[pl.ds(off, 16)]
            xx1 = jnp.maximum(wx1, cx1)
            yy1 = jnp.maximum(wy1, cy1)
            xx2 = jnp.minimum(wx2, cx2)
            yy2 = jnp.minimum(wy2, cy2)
            iw = jnp.maximum(xx2 - xx1 + 1.0, 0.0)
            ih = jnp.maximum(yy2 - yy1 + 1.0, 0.0)
            inter = iw * ih
            carea = (cx2 - cx1 + 1.0) * (cy2 - cy1 + 1.0)
            ovr = inter / (warea + carea - inter)
            s2 = jnp.where(ovr > _NMS_THRESH, _bc_f(_NEG_INF), s)
            sc_v[pl.ds(off, 16)] = s2
            pred = s2 > nbv
            nbv = jnp.where(pred, s2, nbv)
            nbi = jnp.where(pred, _bc_f(off.astype(jnp.float32)) + iota_f, nbi)
            return (nbv, nbi)

        nbv, nbi = lax.fori_loop(
            0, _NVREG, sup_amax,
            (jnp.full((16,), _NEG_INF, jnp.float32),
             jnp.zeros((16,), jnp.float32)))
        cnt_v = cnt_v + jnp.where(alive, _bc_i(jnp.int32(1)),
                                  _bc_i(jnp.int32(0)))
        return (cnt_v, nbv, nbi)

    lax.fori_loop(0, _POST_NMS, round_body,
                  (jnp.zeros((16,), jnp.int32), bv0, bi0))

    @pl.when(wid == 0)
    def _():
        pltpu.sync_copy(out_v, out_hbm)


def _nms_sc(scr, y1, x1, y2, x2, key):
    mesh = plsc.VectorSubcoreMesh(core_axis_name="c", subcore_axis_name="s",
                                  num_cores=1)
    fn = functools.partial(
        pl.kernel, mesh=mesh,
        compiler_params=pltpu.CompilerParams(needs_layout_passes=False),
        out_type=jax.ShapeDtypeStruct((4 * _OUTW,), jnp.float32),
        scratch_types=[
            pltpu.VMEM((_PER_TILE,), jnp.float32),
            pltpu.VMEM((_PER_TILE,), jnp.float32),
            pltpu.VMEM((_PER_TILE,), jnp.float32),
            pltpu.VMEM((_PER_TILE,), jnp.float32),
            pltpu.VMEM((_PER_TILE,), jnp.float32),
            pltpu.VMEM((_PER_TILE,), jnp.int32),
            pltpu.VMEM((4 * _OUTW,), jnp.float32),
            pltpu.VMEM((16,), jnp.float32),
            pltpu.VMEM((16,), jnp.float32),
            pltpu.VMEM((128,), jnp.float32),
            pltpu.VMEM_SHARED((256,), jnp.float32),
        ],
    )(_nms_sc_body)
    return fn(scr, y1, x1, y2, x2, key)


def kernel(features, anchors, W1, b1, Wr, br, Wc, bc):
    dec, keys = _conv_heads(features, W1, b1, Wr, br, Wc, bc, anchors)
    y1 = dec[0:9].reshape(-1)
    x1 = dec[9:18].reshape(-1)
    y2 = dec[18:27].reshape(-1)
    x2 = dec[27:36].reshape(-1)
    scr = dec[36:45].reshape(-1)
    key = keys[0:9].reshape(-1)
    out_flat = _nms_sc(scr, y1, x1, y2, x2, key)
    rois = out_flat.reshape(4, _OUTW)[:, :_POST_NMS].T
    return rois


# speedup vs baseline: 1.4418x; 1.4418x over previous
import functools

import jax
import jax.numpy as jnp
import numpy as np
from jax import lax
from jax.experimental import pallas as pl
from jax.experimental.pallas import tpu as pltpu
from jax.experimental.pallas import tpu_sc as plsc

_F = 50
_P = _F * _F
_PN = 2560
_INC = 512
_K9 = _INC * 9
_NANC = 9
_IMG = 800.0
_MIN_SIZE = 16.0
_NMS_THRESH = 0.7
_PRE_NMS = 6000
_POST_NMS = 300
_NBLK = 512

_NCAND = _NANC * _PN
_NTILES = 16
_PER_TILE = _NCAND // _NTILES
_NVREG = _PER_TILE // 16
_OUTW = 304
_NEG_INF = float("-inf")


def _conv_block(w1_ref, patches_ref, wrc_ref, brc_ref, anc_ref, dec_ref,
                key_ref):
    x = lax.dot_general(
        w1_ref[...], patches_ref[...],
        (((1,), (0,)), ((), ())),
        precision=lax.Precision.DEFAULT,
        preferred_element_type=jnp.float32)
    out = lax.dot_general(
        wrc_ref[...], x,
        (((1,), (0,)), ((), ())),
        precision=lax.Precision.DEFAULT,
        preferred_element_type=jnp.float32)
    out = out + brc_ref[...]

    dy = out[0:9]
    dx = out[9:18]
    dh = out[18:27]
    dw = out[27:36]
    obj = out[36:45]

    y1a = anc_ref[0:9]
    x1a = anc_ref[9:18]
    y2a = anc_ref[18:27]
    x2a = anc_ref[27:36]
    anc_h = y2a - y1a
    anc_w = x2a - x1a
    anc_cy = y1a + 0.5 * anc_h
    anc_cx = x1a + 0.5 * anc_w

    cy = dy * anc_h + anc_cy
    cx = dx * anc_w + anc_cx
    h = jnp.exp(dh) * anc_h
    w = jnp.exp(dw) * anc_w
    y1 = jnp.clip(cy - 0.5 * h, 0.0, _IMG)
    x1 = jnp.clip(cx - 0.5 * w, 0.0, _IMG)
    y2 = jnp.clip(cy + 0.5 * h, 0.0, _IMG)
    x2 = jnp.clip(cx + 0.5 * w, 0.0, _IMG)

    hs = y2 - y1
    ws = x2 - x1
    scr = jnp.where(jnp.logical_and(hs >= _MIN_SIZE, ws >= _MIN_SIZE),
                    obj, _NEG_INF)
    col = pl.program_id(0) * _NBLK + lax.broadcasted_iota(
        jnp.int32, (_NANC, _NBLK), 1)
    scr = jnp.where(col < _P, scr, _NEG_INF)

    dec_ref[0:9] = y1
    dec_ref[9:18] = x1
    dec_ref[18:27] = y2
    dec_ref[27:36] = x2
    dec_ref[36:45] = scr
    dec_ref[45:48] = jnp.zeros((3, _NBLK), jnp.float32)

    bits = lax.bitcast_convert_type(scr, jnp.int32)
    key = jnp.where(bits >= 0, bits,
                    jnp.bitwise_xor(bits, jnp.int32(0x7FFFFFFF)))
    key_ref[0:9] = key
    key_ref[9:16] = jnp.zeros((7, _NBLK), jnp.int32)


def _conv_heads(features, W1, b1, Wr, br, Wc, bc, anchors):
    f = features[0]
    fpad = jnp.pad(f, ((0, 0), (1, 1), (1, 1)))
    taps = [fpad[:, ky:ky + _F, kx:kx + _F].reshape(_INC, _P)
            for ky in range(3) for kx in range(3)]
    patches = jnp.concatenate(taps, axis=0)
    patches = jnp.pad(patches, ((0, 0), (0, _PN - _P)))

    w1mat = jnp.transpose(W1, (0, 2, 3, 1)).reshape(_INC, _K9)

    a = np.arange(_NANC)
    row_sel_r = np.concatenate([4 * a + c for c in range(4)])
    row_sel_c = 2 * a + 1
    wr2 = Wr[:, :, 0, 0]
    wc2 = Wc[:, :, 0, 0]
    wrc = jnp.concatenate([wr2[row_sel_r], wc2[row_sel_c]], axis=0)
    wrc = jnp.pad(wrc, ((0, 64 - 45), (0, 0)))
    bsel = jnp.concatenate([br[row_sel_r], bc[row_sel_c]], axis=0)
    brc = wrc @ b1 + jnp.pad(bsel, (0, 64 - 45))
    brc_b = jnp.broadcast_to(brc[:, None], (64, _PN))

    anc = anchors.reshape(_P, _NANC, 4).transpose(2, 1, 0).reshape(36, _P)
    anc = jnp.pad(anc, ((0, 4), (0, _PN - _P)))

    dec = pl.pallas_call(
        _conv_block,
        grid=(_PN // _NBLK,),
        in_specs=[
            pl.BlockSpec((_INC, _K9), lambda i: (0, 0)),
            pl.BlockSpec((_K9, _NBLK), lambda i: (0, i)),
            pl.BlockSpec((64, _INC), lambda i: (0, 0)),
            pl.BlockSpec((64, _NBLK), lambda i: (0, i)),
            pl.BlockSpec((40, _NBLK), lambda i: (0, i)),
        ],
        out_specs=[pl.BlockSpec((48, _NBLK), lambda i: (0, i)),
                   pl.BlockSpec((16, _NBLK), lambda i: (0, i))],
        out_shape=[jax.ShapeDtypeStruct((48, _PN), jnp.float32),
                   jax.ShapeDtypeStruct((16, _PN), jnp.int32)],
    )(w1mat, patches, wrc, brc_b, anc)
    return dec


def _bc_f(x):
    return lax.broadcast_in_dim(jnp.float32(x) if isinstance(x, float) else x,
                                (16,), ())


def _bc_i(x):
    return lax.broadcast_in_dim(x, (16,), ())


def _nms_sc_body(scr_hbm, y1_hbm, x1_hbm, y2_hbm, x2_hbm, key_hbm, out_hbm,
                 sc_v, y1_v, x1_v, y2_v, x2_v, key_v, out_v, pub_v, tmp_v,
                 stage_v, stage):
    wid = lax.axis_index("s")
    base = wid * _PER_TILE
    iota = lax.iota(jnp.int32, 16)

    pltpu.sync_copy(scr_hbm.at[pl.ds(base, _PER_TILE)], sc_v)
    pltpu.sync_copy(y1_hbm.at[pl.ds(base, _PER_TILE)], y1_v)
    pltpu.sync_copy(x1_hbm.at[pl.ds(base, _PER_TILE)], x1_v)
    pltpu.sync_copy(y2_hbm.at[pl.ds(base, _PER_TILE)], y2_v)
    pltpu.sync_copy(x2_hbm.at[pl.ds(base, _PER_TILE)], x2_v)
    pltpu.sync_copy(key_hbm.at[pl.ds(base, _PER_TILE)], key_v)

    def zero_out(j, carry):
        off = pl.multiple_of(j * 16, 16)
        out_v[pl.ds(off, 16)] = jnp.zeros((16,), jnp.float32)
        return carry

    lax.fori_loop(0, 4 * _OUTW // 16, zero_out, 0)

    iota_f = iota.astype(jnp.float32)

    def allred(vec, op):
        for s in (8, 4, 2, 1):
            tmp_v[pl.ds(0, 16)] = vec
            g = plsc.load_gather(
                tmp_v, [jnp.bitwise_xor(iota, _bc_i(jnp.int32(s)))])
            vec = op(vec, g)
        return vec

    def global_count_ge(t2v, boff):
        def count_vreg(j, acc):
            off = pl.multiple_of(j * 16, 16)
            k = key_v[pl.ds(off, 16)]
            return acc + jnp.where(k >= t2v, _bc_f(jnp.float32(1.0)),
                                   _bc_f(jnp.float32(0.0)))

        acc = lax.fori_loop(0, _NVREG, count_vreg,
                            jnp.zeros((16,), jnp.float32))
        pub_v[pl.ds(0, 16)] = allred(acc, jnp.add)
        pltpu.sync_copy(pub_v.at[pl.ds(0, 8)],
                        stage.at[pl.ds(wid * 8 + boff, 8)])
        plsc.subcore_barrier()
        pltpu.sync_copy(stage.at[pl.ds(boff, 128)], stage_v)
        counts = plsc.load_gather(stage_v, [iota * 8])
        return allred(counts, jnp.add)

    kf = _bc_f(jnp.float32(float(_PRE_NMS)))
    t = jnp.where(global_count_ge(_bc_i(jnp.int32(0)), 0) >= kf,
                  _bc_i(jnp.int32(0)), _bc_i(jnp.int32(-2147483648)))
    for b in range(30, -1, -1):
        t2 = t + _bc_i(jnp.int32(1 << b))
        t = jnp.where(global_count_ge(t2, 128 * ((31 - b) % 2)) >= kf, t2, t)

    def init_live(j, carry):
        off = pl.multiple_of(j * 16, 16)
        k = key_v[pl.ds(off, 16)]
        s = sc_v[pl.ds(off, 16)]
        sc_v[pl.ds(off, 16)] = jnp.where(k >= t, s, _bc_f(_NEG_INF))
        return carry

    lax.fori_loop(0, _NVREG, init_live, 0)
    plsc.subcore_barrier()

    def round_body(r, cnt_v):
        def amax(j, carry):
            bv, bi = carry
            off = pl.multiple_of(j * 16, 16)
            v = sc_v[pl.ds(off, 16)]
            pred = v > bv
            bv = jnp.where(pred, v, bv)
            bi = jnp.where(pred, _bc_f(off.astype(jnp.float32)) + iota_f, bi)
            return (bv, bi)

        bv, bi = lax.fori_loop(
            0, _NVREG, amax,
            (jnp.full((16,), _NEG_INF, jnp.float32),
             jnp.zeros((16,), jnp.float32)))
        mv = allred(bv, jnp.maximum)
        sel = jnp.where(bv >= mv, bi, _bc_f(jnp.float32(float(1 << 30))))
        liv = allred(sel, jnp.minimum).astype(jnp.int32)
        ly1 = plsc.load_gather(y1_v, [liv])
        lx1 = plsc.load_gather(x1_v, [liv])
        ly2 = plsc.load_gather(y2_v, [liv])
        lx2 = plsc.load_gather(x2_v, [liv])
        pub = jnp.where(iota == _bc_i(jnp.int32(0)), mv,
              jnp.where(iota == _bc_i(jnp.int32(1)), ly1,
              jnp.where(iota == _bc_i(jnp.int32(2)), lx1,
              jnp.where(iota == _bc_i(jnp.int32(3)), ly2, lx2))))
        pub_v[pl.ds(0, 16)] = pub
        boff = jnp.bitwise_and(r, jnp.int32(1)) * 128
        pltpu.sync_copy(pub_v.at[pl.ds(0, 8)],
                        stage.at[pl.ds(wid * 8 + boff, 8)])
        plsc.subcore_barrier()
        pltpu.sync_copy(stage.at[pl.ds(boff, 128)], stage_v)

        keys = plsc.load_gather(stage_v, [iota * 8])
        gmaxv = allred(keys, jnp.maximum)
        alive = gmaxv > _bc_f(_NEG_INF)
        wsel = jnp.where(keys >= gmaxv, iota_f, _bc_f(jnp.float32(99.0)))
        woff = allred(wsel, jnp.minimum).astype(jnp.int32) * _bc_i(jnp.int32(8))
        wy1 = plsc.load_gather(stage_v, [woff + _bc_i(jnp.int32(1))])
        wx1 = plsc.load_gather(stage_v, [woff + _bc_i(jnp.int32(2))])
        wy2 = plsc.load_gather(stage_v, [woff + _bc_i(jnp.int32(3))])
        wx2 = plsc.load_gather(stage_v, [woff + _bc_i(jnp.int32(4))])

        outv = jnp.where(iota == _bc_i(jnp.int32(0)), wy1,
               jnp.where(iota == _bc_i(jnp.int32(1)), wx1,
               jnp.where(iota == _bc_i(jnp.int32(2)), wy2, wx2)))
        omask = jnp.logical_and(iota < _bc_i(jnp.int32(4)), alive)
        plsc.store_scatter(out_v, [cnt_v + iota * _OUTW], outv, mask=omask)

        warea = (wx2 - wx1 + 1.0) * (wy2 - wy1 + 1.0)

        def suppress(j, carry):
            off = pl.multiple_of(j * 16, 16)
            cy1 = y1_v[pl.ds(off, 16)]
            cx1 = x1_v[pl.ds(off, 16)]
            cy2 = y2_v[pl.ds(off, 16)]
            cx2 = x2_v[pl.ds(off, 16)]
            s = sc_v[pl.ds(off, 16)]
            xx1 = jnp.maximum(wx1, cx1)
            yy1 = jnp.maximum(wy1, cy1)
            xx2 = jnp.minimum(wx2, cx2)
            yy2 = jnp.minimum(wy2, cy2)
            iw = jnp.maximum(xx2 - xx1 + 1.0, 0.0)
            ih = jnp.maximum(yy2 - yy1 + 1.0, 0.0)
            inter = iw * ih
            carea = (cx2 - cx1 + 1.0) * (cy2 - cy1 + 1.0)
            ovr = inter / (warea + carea - inter)
            sc_v[pl.ds(off, 16)] = jnp.where(ovr > _NMS_THRESH,
                                             _bc_f(_NEG_INF), s)
            return carry

        lax.fori_loop(0, _NVREG, suppress, 0)
        return cnt_v + jnp.where(alive, _bc_i(jnp.int32(1)),
                                 _bc_i(jnp.int32(0)))

    lax.fori_loop(0, _POST_NMS, round_body, jnp.zeros((16,), jnp.int32))

    @pl.when(wid == 0)
    def _():
        pltpu.sync_copy(out_v, out_hbm)


def _nms_sc(scr, y1, x1, y2, x2, key):
    mesh = plsc.VectorSubcoreMesh(core_axis_name="c", subcore_axis_name="s",
                                  num_cores=1)
    fn = functools.partial(
        pl.kernel, mesh=mesh,
        compiler_params=pltpu.CompilerParams(needs_layout_passes=False),
        out_type=jax.ShapeDtypeStruct((4 * _OUTW,), jnp.float32),
        scratch_types=[
            pltpu.VMEM((_PER_TILE,), jnp.float32),
            pltpu.VMEM((_PER_TILE,), jnp.float32),
            pltpu.VMEM((_PER_TILE,), jnp.float32),
            pltpu.VMEM((_PER_TILE,), jnp.float32),
            pltpu.VMEM((_PER_TILE,), jnp.float32),
            pltpu.VMEM((_PER_TILE,), jnp.int32),
            pltpu.VMEM((4 * _OUTW,), jnp.float32),
            pltpu.VMEM((16,), jnp.float32),
            pltpu.VMEM((16,), jnp.float32),
            pltpu.VMEM((128,), jnp.float32),
            pltpu.VMEM_SHARED((256,), jnp.float32),
        ],
    )(_nms_sc_body)
    return fn(scr, y1, x1, y2, x2, key)


def kernel(features, anchors, W1, b1, Wr, br, Wc, bc):
    dec, keys = _conv_heads(features, W1, b1, Wr, br, Wc, bc, anchors)
    y1 = dec[0:9].reshape(-1)
    x1 = dec[9:18].reshape(-1)
    y2 = dec[18:27].reshape(-1)
    x2 = dec[27:36].reshape(-1)
    scr = dec[36:45].reshape(-1)
    key = keys[0:9].reshape(-1)
    out_flat = _nms_sc(scr, y1, x1, y2, x2, key)
    rois = out_flat.reshape(4, _OUTW)[:, :_POST_NMS].T
    return rois


# speedup vs baseline: 1.5852x; 1.0995x over previous
import functools

import jax
import jax.numpy as jnp
import numpy as np
from jax import lax
from jax.experimental import pallas as pl
from jax.experimental.pallas import tpu as pltpu
from jax.experimental.pallas import tpu_sc as plsc

_F = 50
_P = _F * _F
_PN = 2560
_INC = 512
_K9 = _INC * 9
_NANC = 9
_IMG = 800.0
_MIN_SIZE = 16.0
_NMS_THRESH = 0.7
_PRE_NMS = 6000
_POST_NMS = 300
_NBLK = 512

_NCAND = _NANC * _PN
_NTILES = 16
_PER_TILE = _NCAND // _NTILES
_NVREG = _PER_TILE // 16
_OUTW = 304
_NEG_INF = float("-inf")


def _conv_block(w1_ref, patches_ref, wrc_ref, brc_ref, anc_ref, dec_ref,
                key_ref):
    x = lax.dot_general(
        w1_ref[...], patches_ref[...],
        (((1,), (0,)), ((), ())),
        precision=lax.Precision.DEFAULT,
        preferred_element_type=jnp.float32)
    out = lax.dot_general(
        wrc_ref[...], x,
        (((1,), (0,)), ((), ())),
        precision=lax.Precision.DEFAULT,
        preferred_element_type=jnp.float32)
    out = out + brc_ref[...]

    dy = out[0:9]
    dx = out[9:18]
    dh = out[18:27]
    dw = out[27:36]
    obj = out[36:45]

    y1a = anc_ref[0:9]
    x1a = anc_ref[9:18]
    y2a = anc_ref[18:27]
    x2a = anc_ref[27:36]
    anc_h = y2a - y1a
    anc_w = x2a - x1a
    anc_cy = y1a + 0.5 * anc_h
    anc_cx = x1a + 0.5 * anc_w

    cy = dy * anc_h + anc_cy
    cx = dx * anc_w + anc_cx
    h = jnp.exp(dh) * anc_h
    w = jnp.exp(dw) * anc_w
    y1 = jnp.clip(cy - 0.5 * h, 0.0, _IMG)
    x1 = jnp.clip(cx - 0.5 * w, 0.0, _IMG)
    y2 = jnp.clip(cy + 0.5 * h, 0.0, _IMG)
    x2 = jnp.clip(cx + 0.5 * w, 0.0, _IMG)

    hs = y2 - y1
    ws = x2 - x1
    scr = jnp.where(jnp.logical_and(hs >= _MIN_SIZE, ws >= _MIN_SIZE),
                    obj, _NEG_INF)
    col = pl.program_id(0) * _NBLK + lax.broadcasted_iota(
        jnp.int32, (_NANC, _NBLK), 1)
    scr = jnp.where(col < _P, scr, _NEG_INF)

    dec_ref[0:9] = y1
    dec_ref[9:18] = x1
    dec_ref[18:27] = y2
    dec_ref[27:36] = x2
    dec_ref[36:45] = scr
    dec_ref[45:48] = jnp.zeros((3, _NBLK), jnp.float32)

    bits = lax.bitcast_convert_type(scr, jnp.int32)
    key = jnp.where(bits >= 0, bits,
                    jnp.bitwise_xor(bits, jnp.int32(0x7FFFFFFF)))
    key_ref[0:9] = key
    key_ref[9:16] = jnp.zeros((7, _NBLK), jnp.int32)


def _conv_heads(features, W1, b1, Wr, br, Wc, bc, anchors):
    f = features[0]
    fpad = jnp.pad(f, ((0, 0), (1, 1), (1, 1)))
    taps = [fpad[:, ky:ky + _F, kx:kx + _F].reshape(_INC, _P)
            for ky in range(3) for kx in range(3)]
    patches = jnp.concatenate(taps, axis=0)
    patches = jnp.pad(patches, ((0, 0), (0, _PN - _P)))

    w1mat = jnp.transpose(W1, (0, 2, 3, 1)).reshape(_INC, _K9)

    a = np.arange(_NANC)
    row_sel_r = np.concatenate([4 * a + c for c in range(4)])
    row_sel_c = 2 * a + 1
    wr2 = Wr[:, :, 0, 0]
    wc2 = Wc[:, :, 0, 0]
    wrc = jnp.concatenate([wr2[row_sel_r], wc2[row_sel_c]], axis=0)
    wrc = jnp.pad(wrc, ((0, 64 - 45), (0, 0)))
    bsel = jnp.concatenate([br[row_sel_r], bc[row_sel_c]], axis=0)
    brc = wrc @ b1 + jnp.pad(bsel, (0, 64 - 45))
    brc_b = jnp.broadcast_to(brc[:, None], (64, _PN))

    anc = anchors.reshape(_P, _NANC, 4).transpose(2, 1, 0).reshape(36, _P)
    anc = jnp.pad(anc, ((0, 4), (0, _PN - _P)))

    dec = pl.pallas_call(
        _conv_block,
        grid=(_PN // _NBLK,),
        in_specs=[
            pl.BlockSpec((_INC, _K9), lambda i: (0, 0)),
            pl.BlockSpec((_K9, _NBLK), lambda i: (0, i)),
            pl.BlockSpec((64, _INC), lambda i: (0, 0)),
            pl.BlockSpec((64, _NBLK), lambda i: (0, i)),
            pl.BlockSpec((40, _NBLK), lambda i: (0, i)),
        ],
        out_specs=[pl.BlockSpec((48, _NBLK), lambda i: (0, i)),
                   pl.BlockSpec((16, _NBLK), lambda i: (0, i))],
        out_shape=[jax.ShapeDtypeStruct((48, _PN), jnp.float32),
                   jax.ShapeDtypeStruct((16, _PN), jnp.int32)],
    )(w1mat, patches, wrc, brc_b, anc)
    return dec


def _bc_f(x):
    return lax.broadcast_in_dim(jnp.float32(x) if isinstance(x, float) else x,
                                (16,), ())


def _bc_i(x):
    return lax.broadcast_in_dim(x, (16,), ())


def _nms_sc_body(scr_hbm, y1_hbm, x1_hbm, y2_hbm, x2_hbm, key_hbm, out_hbm,
                 sc_v, y1_v, x1_v, y2_v, x2_v, key_v, out_v, pub_v, tmp_v,
                 itmp_v, stage_v, stage):
    wid = lax.axis_index("s")
    base = wid * _PER_TILE
    iota = lax.iota(jnp.int32, 16)

    pltpu.sync_copy(scr_hbm.at[pl.ds(base, _PER_TILE)], sc_v)
    pltpu.sync_copy(y1_hbm.at[pl.ds(base, _PER_TILE)], y1_v)
    pltpu.sync_copy(x1_hbm.at[pl.ds(base, _PER_TILE)], x1_v)
    pltpu.sync_copy(y2_hbm.at[pl.ds(base, _PER_TILE)], y2_v)
    pltpu.sync_copy(x2_hbm.at[pl.ds(base, _PER_TILE)], x2_v)
    pltpu.sync_copy(key_hbm.at[pl.ds(base, _PER_TILE)], key_v)

    def zero_out(j, carry):
        off = pl.multiple_of(j * 16, 16)
        out_v[pl.ds(off, 16)] = jnp.zeros((16,), jnp.float32)
        return carry

    lax.fori_loop(0, 4 * _OUTW // 16, zero_out, 0)

    iota_f = iota.astype(jnp.float32)

    def allred(vec, op):
        for s in (8, 4, 2, 1):
            tmp_v[pl.ds(0, 16)] = vec
            g = plsc.load_gather(
                tmp_v, [jnp.bitwise_xor(iota, _bc_i(jnp.int32(s)))])
            vec = op(vec, g)
        return vec

    def global_count_ge(t2v, boff):
        def count_vreg(j, acc):
            off = pl.multiple_of(j * 16, 16)
            k = key_v[pl.ds(off, 16)]
            return acc + jnp.where(k >= t2v, _bc_f(jnp.float32(1.0)),
                                   _bc_f(jnp.float32(0.0)))

        acc = lax.fori_loop(0, _NVREG, count_vreg,
                            jnp.zeros((16,), jnp.float32))
        pub_v[pl.ds(0, 16)] = allred(acc, jnp.add)
        pltpu.sync_copy(pub_v.at[pl.ds(0, 8)],
                        stage.at[pl.ds(wid * 8 + boff, 8)])
        plsc.subcore_barrier()
        pltpu.sync_copy(stage.at[pl.ds(boff, 128)], stage_v)
        counts = plsc.load_gather(stage_v, [iota * 8])
        return allred(counts, jnp.add)

    kf = _bc_f(jnp.float32(float(_PRE_NMS)))
    t = jnp.where(global_count_ge(_bc_i(jnp.int32(0)), 0) >= kf,
                  _bc_i(jnp.int32(0)), _bc_i(jnp.int32(-2147483648)))
    for b in range(30, -1, -1):
        t2 = t + _bc_i(jnp.int32(1 << b))
        t = jnp.where(global_count_ge(t2, 128 * ((31 - b) % 2)) >= kf, t2, t)

    def compact(j, w_vec):
        off = pl.multiple_of(j * 16, 16)
        k = key_v[pl.ds(off, 16)]
        s = sc_v[pl.ds(off, 16)]
        cy1 = y1_v[pl.ds(off, 16)]
        cx1 = x1_v[pl.ds(off, 16)]
        cy2 = y2_v[pl.ds(off, 16)]
        cx2 = x2_v[pl.ds(off, 16)]
        mask = k >= t
        mf = jnp.where(mask, _bc_f(jnp.float32(1.0)), _bc_f(jnp.float32(0.0)))
        v = mf
        for sh in (1, 2, 4, 8):
            tmp_v[pl.ds(0, 16)] = v
            idx = iota - _bc_i(jnp.int32(sh))
            g = plsc.load_gather(tmp_v,
                                 [jnp.maximum(idx, _bc_i(jnp.int32(0)))])
            v = v + jnp.where(iota >= _bc_i(jnp.int32(sh)), g,
                              _bc_f(jnp.float32(0.0)))
        tmp_v[pl.ds(0, 16)] = v
        tot = plsc.load_gather(tmp_v, [_bc_i(jnp.int32(15))])
        widx = w_vec + (v - mf).astype(jnp.int32)
        plsc.store_scatter(sc_v, [widx], s, mask=mask)
        plsc.store_scatter(y1_v, [widx], cy1, mask=mask)
        plsc.store_scatter(x1_v, [widx], cx1, mask=mask)
        plsc.store_scatter(y2_v, [widx], cy2, mask=mask)
        plsc.store_scatter(x2_v, [widx], cx2, mask=mask)
        return w_vec + tot.astype(jnp.int32)

    w_vec = lax.fori_loop(0, _NVREG, compact, jnp.zeros((16,), jnp.int32))
    w0 = w_vec[0]
    nv = lax.shift_right_logical(w0 + 15, 4)
    vbase = lax.shift_left(lax.shift_right_logical(w0, 4), 4)

    @pl.when(vbase < _PER_TILE)
    def _():
        cur = sc_v[pl.ds(vbase, 16)]
        sc_v[pl.ds(vbase, 16)] = jnp.where(_bc_i(vbase) + iota >= _bc_i(w0),
                                           _bc_f(_NEG_INF), cur)

    plsc.subcore_barrier()

    def round_body(r, cnt_v):
        def amax(j, carry):
            bv, bi = carry
            off = pl.multiple_of(j * 16, 16)
            v = sc_v[pl.ds(off, 16)]
            pred = v > bv
            bv = jnp.where(pred, v, bv)
            bi = jnp.where(pred, _bc_f(off.astype(jnp.float32)) + iota_f, bi)
            return (bv, bi)

        bv, bi = lax.fori_loop(
            0, nv, amax,
            (jnp.full((16,), _NEG_INF, jnp.float32),
             jnp.zeros((16,), jnp.float32)))
        mv = allred(bv, jnp.maximum)
        sel = jnp.where(bv >= mv, bi, _bc_f(jnp.float32(float(1 << 30))))
        liv = allred(sel, jnp.minimum).astype(jnp.int32)
        ly1 = plsc.load_gather(y1_v, [liv])
        lx1 = plsc.load_gather(x1_v, [liv])
        ly2 = plsc.load_gather(y2_v, [liv])
        lx2 = plsc.load_gather(x2_v, [liv])
        pub = jnp.where(iota == _bc_i(jnp.int32(0)), mv,
              jnp.where(iota == _bc_i(jnp.int32(1)), ly1,
              jnp.where(iota == _bc_i(jnp.int32(2)), lx1,
              jnp.where(iota == _bc_i(jnp.int32(3)), ly2, lx2))))
        pub_v[pl.ds(0, 16)] = pub
        boff = jnp.bitwise_and(r, jnp.int32(1)) * 128
        pltpu.sync_copy(pub_v.at[pl.ds(0, 8)],
                        stage.at[pl.ds(wid * 8 + boff, 8)])
        plsc.subcore_barrier()
        pltpu.sync_copy(stage.at[pl.ds(boff, 128)], stage_v)

        keys = plsc.load_gather(stage_v, [iota * 8])
        gmaxv = allred(keys, jnp.maximum)
        alive = gmaxv > _bc_f(_NEG_INF)
        wsel = jnp.where(keys >= gmaxv, iota_f, _bc_f(jnp.float32(99.0)))
        woff = allred(wsel, jnp.minimum).astype(jnp.int32) * _bc_i(jnp.int32(8))
        wy1 = plsc.load_gather(stage_v, [woff + _bc_i(jnp.int32(1))])
        wx1 = plsc.load_gather(stage_v, [woff + _bc_i(jnp.int32(2))])
        wy2 = plsc.load_gather(stage_v, [woff + _bc_i(jnp.int32(3))])
        wx2 = plsc.load_gather(stage_v, [woff + _bc_i(jnp.int32(4))])

        outv = jnp.where(iota == _bc_i(jnp.int32(0)), wy1,
               jnp.where(iota == _bc_i(jnp.int32(1)), wx1,
               jnp.where(iota == _bc_i(jnp.int32(2)), wy2, wx2)))
        omask = jnp.logical_and(iota < _bc_i(jnp.int32(4)), alive)
        plsc.store_scatter(out_v, [cnt_v + iota * _OUTW], outv, mask=omask)

        warea = (wx2 - wx1 + 1.0) * (wy2 - wy1 + 1.0)

        def suppress(j, carry):
            off = pl.multiple_of(j * 16, 16)
            cy1 = y1_v[pl.ds(off, 16)]
            cx1 = x1_v[pl.ds(off, 16)]
            cy2 = y2_v[pl.ds(off, 16)]
            cx2 = x2_v[pl.ds(off, 16)]
            s = sc_v[pl.ds(off, 16)]
            xx1 = jnp.maximum(wx1, cx1)
            yy1 = jnp.maximum(wy1, cy1)
            xx2 = jnp.minimum(wx2, cx2)
            yy2 = jnp.minimum(wy2, cy2)
            iw = jnp.maximum(xx2 - xx1 + 1.0, 0.0)
            ih = jnp.maximum(yy2 - yy1 + 1.0, 0.0)
            inter = iw * ih
            carea = (cx2 - cx1 + 1.0) * (cy2 - cy1 + 1.0)
            ovr = inter / (warea + carea - inter)
            sc_v[pl.ds(off, 16)] = jnp.where(ovr > _NMS_THRESH,
                                             _bc_f(_NEG_INF), s)
            return carry

        lax.fori_loop(0, nv, suppress, 0)
        return cnt_v + jnp.where(alive, _bc_i(jnp.int32(1)),
                                 _bc_i(jnp.int32(0)))

    lax.fori_loop(0, _POST_NMS, round_body, jnp.zeros((16,), jnp.int32))

    @pl.when(wid == 0)
    def _():
        pltpu.sync_copy(out_v, out_hbm)


def _nms_sc(scr, y1, x1, y2, x2, key):
    mesh = plsc.VectorSubcoreMesh(core_axis_name="c", subcore_axis_name="s",
                                  num_cores=1)
    fn = functools.partial(
        pl.kernel, mesh=mesh,
        compiler_params=pltpu.CompilerParams(needs_layout_passes=False),
        out_type=jax.ShapeDtypeStruct((4 * _OUTW,), jnp.float32),
        scratch_types=[
            pltpu.VMEM((_PER_TILE,), jnp.float32),
            pltpu.VMEM((_PER_TILE,), jnp.float32),
            pltpu.VMEM((_PER_TILE,), jnp.float32),
            pltpu.VMEM((_PER_TILE,), jnp.float32),
            pltpu.VMEM((_PER_TILE,), jnp.float32),
            pltpu.VMEM((_PER_TILE,), jnp.int32),
            pltpu.VMEM((4 * _OUTW,), jnp.float32),
            pltpu.VMEM((16,), jnp.float32),
            pltpu.VMEM((16,), jnp.float32),
            pltpu.VMEM((16,), jnp.int32),
            pltpu.VMEM((128,), jnp.float32),
            pltpu.VMEM_SHARED((256,), jnp.float32),
        ],
    )(_nms_sc_body)
    return fn(scr, y1, x1, y2, x2, key)


def kernel(features, anchors, W1, b1, Wr, br, Wc, bc):
    dec, keys = _conv_heads(features, W1, b1, Wr, br, Wc, bc, anchors)
    y1 = dec[0:9].reshape(-1)
    x1 = dec[9:18].reshape(-1)
    y2 = dec[18:27].reshape(-1)
    x2 = dec[27:36].reshape(-1)
    scr = dec[36:45].reshape(-1)
    key = keys[0:9].reshape(-1)
    out_flat = _nms_sc(scr, y1, x1, y2, x2, key)
    rois = out_flat.reshape(4, _OUTW)[:, :_POST_NMS].T
    return rois


# speedup vs baseline: 2.1010x; 1.3254x over previous
import functools

import jax
import jax.numpy as jnp
import numpy as np
from jax import lax
from jax.experimental import pallas as pl
from jax.experimental.pallas import tpu as pltpu
from jax.experimental.pallas import tpu_sc as plsc

_F = 50
_FP = 52
_Q = _FP * _FP
_QB = 3072
_QPAD = 3584
_INC = 512
_K9 = _INC * 9
_NANC = 9
_IMG = 800.0
_MIN_SIZE = 16.0
_NMS_THRESH = 0.7
_PRE_NMS = 6000
_POST_NMS = 300
_NBLK = 512
_OFFS = [(ky - 1) * _FP + (kx - 1) for ky in range(3) for kx in range(3)]

_NCAND = _NANC * _QB
_NTILES = 16
_PER_TILE = _NCAND // _NTILES
_NVREG = _PER_TILE // 16
_OUTW = 304
_NEG_INF = float("-inf")


def _conv_block(w1_ref, fpad_ref, wrc_ref, brc_ref, anc_ref, dec_ref,
                key_ref):
    i = pl.program_id(0)
    win = fpad_ref[:, pl.ds(i * _NBLK, 1024)]
    patches = jnp.concatenate(
        [win[:, 64 + off:64 + off + _NBLK] for off in _OFFS], axis=0)
    x = lax.dot_general(
        w1_ref[...], patches,
        (((1,), (0,)), ((), ())),
        precision=lax.Precision.DEFAULT,
        preferred_element_type=jnp.float32)
    out = lax.dot_general(
        wrc_ref[...], x,
        (((1,), (0,)), ((), ())),
        precision=lax.Precision.DEFAULT,
        preferred_element_type=jnp.float32)
    out = out + brc_ref[...]

    dy = out[0:9]
    dx = out[9:18]
    dh = out[18:27]
    dw = out[27:36]
    obj = out[36:45]

    y1a = anc_ref[0:9]
    x1a = anc_ref[9:18]
    y2a = anc_ref[18:27]
    x2a = anc_ref[27:36]
    anc_h = y2a - y1a
    anc_w = x2a - x1a
    anc_cy = y1a + 0.5 * anc_h
    anc_cx = x1a + 0.5 * anc_w

    cy = dy * anc_h + anc_cy
    cx = dx * anc_w + anc_cx
    h = jnp.exp(dh) * anc_h
    w = jnp.exp(dw) * anc_w
    y1 = jnp.clip(cy - 0.5 * h, 0.0, _IMG)
    x1 = jnp.clip(cx - 0.5 * w, 0.0, _IMG)
    y2 = jnp.clip(cy + 0.5 * h, 0.0, _IMG)
    x2 = jnp.clip(cx + 0.5 * w, 0.0, _IMG)

    hs = y2 - y1
    ws = x2 - x1
    scr = jnp.where(jnp.logical_and(hs >= _MIN_SIZE, ws >= _MIN_SIZE),
                    obj, _NEG_INF)
    qv = pl.program_id(0) * _NBLK + lax.broadcasted_iota(
        jnp.int32, (_NANC, _NBLK), 1)
    iy = qv // _FP
    ix = qv % _FP
    valid = jnp.logical_and(
        jnp.logical_and(qv < _Q, jnp.logical_and(iy >= 1, iy <= _F)),
        jnp.logical_and(ix >= 1, ix <= _F))
    scr = jnp.where(valid, scr, _NEG_INF)

    dec_ref[0:9] = y1
    dec_ref[9:18] = x1
    dec_ref[18:27] = y2
    dec_ref[27:36] = x2
    dec_ref[36:45] = scr
    dec_ref[45:48] = jnp.zeros((3, _NBLK), jnp.float32)

    bits = lax.bitcast_convert_type(scr, jnp.int32)
    key = jnp.where(bits >= 0, bits,
                    jnp.bitwise_xor(bits, jnp.int32(0x7FFFFFFF)))
    key_ref[0:9] = key
    key_ref[9:16] = jnp.zeros((7, _NBLK), jnp.int32)


def _conv_heads(features, W1, b1, Wr, br, Wc, bc, anchors):
    f = features[0]
    fpad = jnp.pad(f, ((0, 0), (1, 1), (1, 1)))
    fpad2 = jnp.pad(fpad.reshape(_INC, _Q),
                    ((0, 0), (64, _QPAD - 64 - _Q)))

    w1mat = jnp.transpose(W1, (0, 2, 3, 1)).reshape(_INC, _K9)

    a = np.arange(_NANC)
    row_sel_r = np.concatenate([4 * a + c for c in range(4)])
    row_sel_c = 2 * a + 1
    wr2 = Wr[:, :, 0, 0]
    wc2 = Wc[:, :, 0, 0]
    wrc = jnp.concatenate([wr2[row_sel_r], wc2[row_sel_c]], axis=0)
    wrc = jnp.pad(wrc, ((0, 64 - 45), (0, 0)))
    bsel = jnp.concatenate([br[row_sel_r], bc[row_sel_c]], axis=0)
    brc = wrc @ b1 + jnp.pad(bsel, (0, 64 - 45))
    brc_b = jnp.broadcast_to(brc[:, None], (64, _QB))

    anc = anchors.reshape(_F * _F, _NANC, 4).transpose(2, 1, 0)
    anc = anc.reshape(36, _F, _F)
    anc = jnp.pad(anc, ((0, 0), (1, 1), (1, 1))).reshape(36, _Q)
    anc = jnp.pad(anc, ((0, 4), (0, _QB - _Q)))

    dec = pl.pallas_call(
        _conv_block,
        grid=(_QB // _NBLK,),
        in_specs=[
            pl.BlockSpec((_INC, _K9), lambda i: (0, 0)),
            pl.BlockSpec((_INC, _QPAD), lambda i: (0, 0)),
            pl.BlockSpec((64, _INC), lambda i: (0, 0)),
            pl.BlockSpec((64, _NBLK), lambda i: (0, i)),
            pl.BlockSpec((40, _NBLK), lambda i: (0, i)),
        ],
        out_specs=[pl.BlockSpec((48, _NBLK), lambda i: (0, i)),
                   pl.BlockSpec((16, _NBLK), lambda i: (0, i))],
        out_shape=[jax.ShapeDtypeStruct((48, _QB), jnp.float32),
                   jax.ShapeDtypeStruct((16, _QB), jnp.int32)],
    )(w1mat, fpad2, wrc, brc_b, anc)
    return dec


def _bc_f(x):
    return lax.broadcast_in_dim(jnp.float32(x) if isinstance(x, float) else x,
                                (16,), ())


def _bc_i(x):
    return lax.broadcast_in_dim(x, (16,), ())


def _nms_sc_body(scr_hbm, y1_hbm, x1_hbm, y2_hbm, x2_hbm, key_hbm, out_hbm,
                 sc_v, y1_v, x1_v, y2_v, x2_v, key_v, out_v, pub_v, tmp_v,
                 itmp_v, stage_v, stage):
    wid = lax.axis_index("s")
    base = wid * _PER_TILE
    iota = lax.iota(jnp.int32, 16)

    pltpu.sync_copy(scr_hbm.at[pl.ds(base, _PER_TILE)], sc_v)
    pltpu.sync_copy(y1_hbm.at[pl.ds(base, _PER_TILE)], y1_v)
    pltpu.sync_copy(x1_hbm.at[pl.ds(base, _PER_TILE)], x1_v)
    pltpu.sync_copy(y2_hbm.at[pl.ds(base, _PER_TILE)], y2_v)
    pltpu.sync_copy(x2_hbm.at[pl.ds(base, _PER_TILE)], x2_v)
    pltpu.sync_copy(key_hbm.at[pl.ds(base, _PER_TILE)], key_v)

    def zero_out(j, carry):
        off = pl.multiple_of(j * 16, 16)
        out_v[pl.ds(off, 16)] = jnp.zeros((16,), jnp.float32)
        return carry

    lax.fori_loop(0, 4 * _OUTW // 16, zero_out, 0)

    iota_f = iota.astype(jnp.float32)

    def allred(vec, op):
        for s in (8, 4, 2, 1):
            tmp_v[pl.ds(0, 16)] = vec
            g = plsc.load_gather(
                tmp_v, [jnp.bitwise_xor(iota, _bc_i(jnp.int32(s)))])
            vec = op(vec, g)
        return vec

    def global_count_ge(t2v, boff):
        def count_vreg(j, acc):
            off = pl.multiple_of(j * 16, 16)
            k = key_v[pl.ds(off, 16)]
            return acc + jnp.where(k >= t2v, _bc_f(jnp.float32(1.0)),
                                   _bc_f(jnp.float32(0.0)))

        acc = lax.fori_loop(0, _NVREG, count_vreg,
                            jnp.zeros((16,), jnp.float32))
        pub_v[pl.ds(0, 16)] = allred(acc, jnp.add)
        pltpu.sync_copy(pub_v.at[pl.ds(0, 8)],
                        stage.at[pl.ds(wid * 8 + boff, 8)])
        plsc.subcore_barrier()
        pltpu.sync_copy(stage.at[pl.ds(boff, 128)], stage_v)
        counts = plsc.load_gather(stage_v, [iota * 8])
        return allred(counts, jnp.add)

    kf = _bc_f(jnp.float32(float(_PRE_NMS)))
    t = jnp.where(global_count_ge(_bc_i(jnp.int32(0)), 0) >= kf,
                  _bc_i(jnp.int32(0)), _bc_i(jnp.int32(-2147483648)))
    for b in range(30, -1, -1):
        t2 = t + _bc_i(jnp.int32(1 << b))
        t = jnp.where(global_count_ge(t2, 128 * ((31 - b) % 2)) >= kf, t2, t)

    def compact(j, w_vec):
        off = pl.multiple_of(j * 16, 16)
        k = key_v[pl.ds(off, 16)]
        s = sc_v[pl.ds(off, 16)]
        cy1 = y1_v[pl.ds(off, 16)]
        cx1 = x1_v[pl.ds(off, 16)]
        cy2 = y2_v[pl.ds(off, 16)]
        cx2 = x2_v[pl.ds(off, 16)]
        mask = k >= t
        mf = jnp.where(mask, _bc_f(jnp.float32(1.0)), _bc_f(jnp.float32(0.0)))
        v = mf
        for sh in (1, 2, 4, 8):
            tmp_v[pl.ds(0, 16)] = v
            idx = iota - _bc_i(jnp.int32(sh))
            g = plsc.load_gather(tmp_v,
                                 [jnp.maximum(idx, _bc_i(jnp.int32(0)))])
            v = v + jnp.where(iota >= _bc_i(jnp.int32(sh)), g,
                              _bc_f(jnp.float32(0.0)))
        tmp_v[pl.ds(0, 16)] = v
        tot = plsc.load_gather(tmp_v, [_bc_i(jnp.int32(15))])
        widx = w_vec + (v - mf).astype(jnp.int32)
        plsc.store_scatter(sc_v, [widx], s, mask=mask)
        plsc.store_scatter(y1_v, [widx], cy1, mask=mask)
        plsc.store_scatter(x1_v, [widx], cx1, mask=mask)
        plsc.store_scatter(y2_v, [widx], cy2, mask=mask)
        plsc.store_scatter(x2_v, [widx], cx2, mask=mask)
        return w_vec + tot.astype(jnp.int32)

    w_vec = lax.fori_loop(0, _NVREG, compact, jnp.zeros((16,), jnp.int32))
    w0 = w_vec[0]
    nv = lax.shift_right_logical(w0 + 15, 4)
    vbase = lax.shift_left(lax.shift_right_logical(w0, 4), 4)

    @pl.when(vbase < _PER_TILE)
    def _():
        cur = sc_v[pl.ds(vbase, 16)]
        sc_v[pl.ds(vbase, 16)] = jnp.where(_bc_i(vbase) + iota >= _bc_i(w0),
                                           _bc_f(_NEG_INF), cur)

    plsc.subcore_barrier()

    def round_body(r, cnt_v):
        def amax(j, carry):
            bv, bi = carry
            off = pl.multiple_of(j * 16, 16)
            v = sc_v[pl.ds(off, 16)]
            pred = v > bv
            bv = jnp.where(pred, v, bv)
            bi = jnp.where(pred, _bc_f(off.astype(jnp.float32)) + iota_f, bi)
            return (bv, bi)

        bv, bi = lax.fori_loop(
            0, nv, amax,
            (jnp.full((16,), _NEG_INF, jnp.float32),
             jnp.zeros((16,), jnp.float32)))
        mv = allred(bv, jnp.maximum)
        sel = jnp.where(bv >= mv, bi, _bc_f(jnp.float32(float(1 << 30))))
        liv = allred(sel, jnp.minimum).astype(jnp.int32)
        ly1 = plsc.load_gather(y1_v, [liv])
        lx1 = plsc.load_gather(x1_v, [liv])
        ly2 = plsc.load_gather(y2_v, [liv])
        lx2 = plsc.load_gather(x2_v, [liv])
        pub = jnp.where(iota == _bc_i(jnp.int32(0)), mv,
              jnp.where(iota == _bc_i(jnp.int32(1)), ly1,
              jnp.where(iota == _bc_i(jnp.int32(2)), lx1,
              jnp.where(iota == _bc_i(jnp.int32(3)), ly2, lx2))))
        pub_v[pl.ds(0, 16)] = pub
        boff = jnp.bitwise_and(r, jnp.int32(1)) * 128
        pltpu.sync_copy(pub_v.at[pl.ds(0, 8)],
                        stage.at[pl.ds(wid * 8 + boff, 8)])
        plsc.subcore_barrier()
        pltpu.sync_copy(stage.at[pl.ds(boff, 128)], stage_v)

        keys = plsc.load_gather(stage_v, [iota * 8])
        gmaxv = allred(keys, jnp.maximum)
        alive = gmaxv > _bc_f(_NEG_INF)
        wsel = jnp.where(keys >= gmaxv, iota_f, _bc_f(jnp.float32(99.0)))
        woff = allred(wsel, jnp.minimum).astype(jnp.int32) * _bc_i(jnp.int32(8))
        wy1 = plsc.load_gather(stage_v, [woff + _bc_i(jnp.int32(1))])
        wx1 = plsc.load_gather(stage_v, [woff + _bc_i(jnp.int32(2))])
        wy2 = plsc.load_gather(stage_v, [woff + _bc_i(jnp.int32(3))])
        wx2 = plsc.load_gather(stage_v, [woff + _bc_i(jnp.int32(4))])

        outv = jnp.where(iota == _bc_i(jnp.int32(0)), wy1,
               jnp.where(iota == _bc_i(jnp.int32(1)), wx1,
               jnp.where(iota == _bc_i(jnp.int32(2)), wy2, wx2)))
        omask = jnp.logical_and(iota < _bc_i(jnp.int32(4)), alive)
        plsc.store_scatter(out_v, [cnt_v + iota * _OUTW], outv, mask=omask)

        warea = (wx2 - wx1 + 1.0) * (wy2 - wy1 + 1.0)

        def suppress(j, carry):
            off = pl.multiple_of(j * 16, 16)
            cy1 = y1_v[pl.ds(off, 16)]
            cx1 = x1_v[pl.ds(off, 16)]
            cy2 = y2_v[pl.ds(off, 16)]
            cx2 = x2_v[pl.ds(off, 16)]
            s = sc_v[pl.ds(off, 16)]
            xx1 = jnp.maximum(wx1, cx1)
            yy1 = jnp.maximum(wy1, cy1)
            xx2 = jnp.minimum(wx2, cx2)
            yy2 = jnp.minimum(wy2, cy2)
            iw = jnp.maximum(xx2 - xx1 + 1.0, 0.0)
            ih = jnp.maximum(yy2 - yy1 + 1.0, 0.0)
            inter = iw * ih
            carea = (cx2 - cx1 + 1.0) * (cy2 - cy1 + 1.0)
            ovr = inter / (warea + carea - inter)
            sc_v[pl.ds(off, 16)] = jnp.where(ovr > _NMS_THRESH,
                                             _bc_f(_NEG_INF), s)
            return carry

        lax.fori_loop(0, nv, suppress, 0)
        return cnt_v + jnp.where(alive, _bc_i(jnp.int32(1)),
                                 _bc_i(jnp.int32(0)))

    lax.fori_loop(0, _POST_NMS, round_body, jnp.zeros((16,), jnp.int32))

    @pl.when(wid == 0)
    def _():
        pltpu.sync_copy(out_v, out_hbm)


def _nms_sc(scr, y1, x1, y2, x2, key):
    mesh = plsc.VectorSubcoreMesh(core_axis_name="c", subcore_axis_name="s",
                                  num_cores=1)
    fn = functools.partial(
        pl.kernel, mesh=mesh,
        compiler_params=pltpu.CompilerParams(needs_layout_passes=False),
        out_type=jax.ShapeDtypeStruct((4 * _OUTW,), jnp.float32),
        scratch_types=[
            pltpu.VMEM((_PER_TILE,), jnp.float32),
            pltpu.VMEM((_PER_TILE,), jnp.float32),
            pltpu.VMEM((_PER_TILE,), jnp.float32),
            pltpu.VMEM((_PER_TILE,), jnp.float32),
            pltpu.VMEM((_PER_TILE,), jnp.float32),
            pltpu.VMEM((_PER_TILE,), jnp.int32),
            pltpu.VMEM((4 * _OUTW,), jnp.float32),
            pltpu.VMEM((16,), jnp.float32),
            pltpu.VMEM((16,), jnp.float32),
            pltpu.VMEM((16,), jnp.int32),
            pltpu.VMEM((128,), jnp.float32),
            pltpu.VMEM_SHARED((256,), jnp.float32),
        ],
    )(_nms_sc_body)
    return fn(scr, y1, x1, y2, x2, key)


def kernel(features, anchors, W1, b1, Wr, br, Wc, bc):
    dec, keys = _conv_heads(features, W1, b1, Wr, br, Wc, bc, anchors)
    y1 = dec[0:9].reshape(-1)
    x1 = dec[9:18].reshape(-1)
    y2 = dec[18:27].reshape(-1)
    x2 = dec[27:36].reshape(-1)
    scr = dec[36:45].reshape(-1)
    key = keys[0:9].reshape(-1)
    out_flat = _nms_sc(scr, y1, x1, y2, x2, key)
    rois = out_flat.reshape(4, _OUTW)[:, :_POST_NMS].T
    return rois
